# Initial kernel scaffold; baseline (speedup 1.0000x reference)
import jax
import jax.numpy as jnp
from jax.experimental import pallas as pl


def kernel(x, edge_index, W1, b1, W2, b2, Wmu, bmu, Wlv, blv):
    raise NotImplementedError("write your pallas kernel here")


# speedup vs baseline: 11.1106x; 11.1106x over previous
import functools

import jax
import jax.numpy as jnp
from jax import lax
from jax.experimental import pallas as pl
from jax.experimental.pallas import tpu as pltpu
from jax.experimental.pallas import tpu_sc as plsc

N = 10000
E = 160000
D_IN = 256
D_H = 256
D_Z = 64

NC = 2
NS = 16
CHUNK = 128
NCHUNKS = E // CHUNK
ROWS_PER_TILE = 624
TAIL_ROWS = N - NS * ROWS_PER_TILE
DEG_PAD = NS * 640

@functools.cache
def _mesh():
    return plsc.VectorSubcoreMesh(
        core_axis_name="c", subcore_axis_name="s",
        num_cores=NC, num_subcores=NS)


def _fill(ref, n, value, dtype):
    v = jnp.full((16,), value, dtype=dtype)
    for i in range(n // 16):
        ref[pl.ds(i * 16, 16)] = v


def _deg_body(edge_ref, deg_out, idx_v, ones_v, zeros_v, deg_sh):
    c = lax.axis_index("c")
    t = lax.axis_index("s")
    _fill(ones_v, CHUNK, 1.0, jnp.float32)
    _fill(zeros_v, 640, 0.0, jnp.float32)
    pltpu.sync_copy(zeros_v, deg_sh.at[pl.ds(t * 640, 640)])
    plsc.subcore_barrier()

    half = NCHUNKS // NC
    base = c * half
    n_k = (half - t + NS - 1) // NS

    def body(k, carry):
        j = base + t + NS * k
        off = j * CHUNK
        pltpu.sync_copy(edge_ref.at[pl.ds(E + off, CHUNK)], idx_v)
        pltpu.sync_copy(ones_v, deg_sh.at[idx_v], add=True)
        return carry

    lax.fori_loop(0, n_k, body, 0)
    plsc.subcore_barrier()
    pltpu.sync_copy(deg_sh.at[pl.ds(t * 640, 640)],
                    deg_out.at[pl.ds(c * DEG_PAD + t * 640, 640)])


@functools.cache
def _deg_call():
    return pl.kernel(
        _deg_body,
        out_type=jax.ShapeDtypeStruct((NC * DEG_PAD,), jnp.float32),
        mesh=_mesh(),
        scratch_types=[
            pltpu.VMEM((CHUNK,), jnp.int32),
            pltpu.VMEM((CHUNK,), jnp.float32),
            pltpu.VMEM((640,), jnp.float32),
            pltpu.VMEM_SHARED((DEG_PAD,), jnp.float32),
        ],
    )


def _prop_core(h, o, edge_ref, idx_s, idx_d, rows, acc, gsem, t):
    r0 = t * ROWS_PER_TILE
    pltpu.sync_copy(h.at[pl.ds(r0, ROWS_PER_TILE)],
                    acc.at[pl.ds(r0, ROWS_PER_TILE)])

    @pl.when(t == NS - 1)
    def _():
        pltpu.sync_copy(h.at[pl.ds(NS * ROWS_PER_TILE, TAIL_ROWS)],
                        acc.at[pl.ds(NS * ROWS_PER_TILE, TAIL_ROWS)])

    plsc.subcore_barrier()

    n_k = (NCHUNKS - t + NS - 1) // NS

    def body(k, carry):
        off = (t + NS * k) * CHUNK
        pltpu.sync_copy(edge_ref.at[pl.ds(off, CHUNK)], idx_s)
        pltpu.sync_copy(edge_ref.at[pl.ds(E + off, CHUNK)], idx_d)
        pltpu.async_copy(h.at[idx_s], rows, gsem).wait()
        pltpu.sync_copy(rows, acc.at[idx_d], add=True)
        return carry

    lax.fori_loop(0, n_k, body, 0)
    plsc.subcore_barrier()
    pltpu.sync_copy(acc.at[pl.ds(r0, ROWS_PER_TILE)],
                    o.at[pl.ds(r0, ROWS_PER_TILE)])

    @pl.when(t == NS - 1)
    def _():
        pltpu.sync_copy(acc.at[pl.ds(NS * ROWS_PER_TILE, TAIL_ROWS)],
                        o.at[pl.ds(NS * ROWS_PER_TILE, TAIL_ROWS)])


def _prop_body(h0, h1, edge_ref, o0, o1, idx_s, idx_d, rows, acc, gsem):
    c = lax.axis_index("c")
    t = lax.axis_index("s")

    @pl.when(c == 0)
    def _():
        _prop_core(h0, o0, edge_ref, idx_s, idx_d, rows, acc, gsem, t)

    @pl.when(c == 1)
    def _():
        _prop_core(h1, o1, edge_ref, idx_s, idx_d, rows, acc, gsem, t)


def _prop_half_body(v, edge_ref, o0, o1, idx_s, idx_d, rows, acc, gsem):
    c = lax.axis_index("c")
    t = lax.axis_index("s")
    r0 = t * ROWS_PER_TILE
    pltpu.sync_copy(v.at[pl.ds(r0, ROWS_PER_TILE)],
                    acc.at[pl.ds(r0, ROWS_PER_TILE)])

    @pl.when(t == NS - 1)
    def _():
        pltpu.sync_copy(v.at[pl.ds(NS * ROWS_PER_TILE, TAIL_ROWS)],
                        acc.at[pl.ds(NS * ROWS_PER_TILE, TAIL_ROWS)])

    plsc.subcore_barrier()

    half = NCHUNKS // NC
    base = c * half
    n_k = (half - t + NS - 1) // NS

    def body(k, carry):
        off = (base + t + NS * k) * CHUNK
        pltpu.sync_copy(edge_ref.at[pl.ds(off, CHUNK)], idx_s)
        pltpu.sync_copy(edge_ref.at[pl.ds(E + off, CHUNK)], idx_d)
        pltpu.async_copy(v.at[idx_s], rows, gsem).wait()
        pltpu.sync_copy(rows, acc.at[idx_d], add=True)
        return carry

    lax.fori_loop(0, n_k, body, 0)
    plsc.subcore_barrier()

    def writeout(o):
        pltpu.sync_copy(acc.at[pl.ds(r0, ROWS_PER_TILE)],
                        o.at[pl.ds(r0, ROWS_PER_TILE)])

        @pl.when(t == NS - 1)
        def _():
            pltpu.sync_copy(acc.at[pl.ds(NS * ROWS_PER_TILE, TAIL_ROWS)],
                            o.at[pl.ds(NS * ROWS_PER_TILE, TAIL_ROWS)])

    @pl.when(c == 0)
    def _():
        writeout(o0)

    @pl.when(c == 1)
    def _():
        writeout(o1)


@functools.cache
def _prop_half_call():
    return pl.kernel(
        _prop_half_body,
        out_type=(jax.ShapeDtypeStruct((N, 2 * D_Z), jnp.float32),
                  jax.ShapeDtypeStruct((N, 2 * D_Z), jnp.float32)),
        mesh=_mesh(),
        scratch_types=[
            pltpu.VMEM((CHUNK,), jnp.int32),
            pltpu.VMEM((CHUNK,), jnp.int32),
            pltpu.VMEM((CHUNK, 2 * D_Z), jnp.float32),
            pltpu.VMEM_SHARED((N, 2 * D_Z), jnp.float32),
            pltpu.SemaphoreType.DMA,
        ],
    )


@functools.cache
def _prop_call(w):
    return pl.kernel(
        _prop_body,
        out_type=(jax.ShapeDtypeStruct((N, w), jnp.float32),
                  jax.ShapeDtypeStruct((N, w), jnp.float32)),
        mesh=_mesh(),
        scratch_types=[
            pltpu.VMEM((CHUNK,), jnp.int32),
            pltpu.VMEM((CHUNK,), jnp.int32),
            pltpu.VMEM((CHUNK, w), jnp.float32),
            pltpu.VMEM_SHARED((N, w), jnp.float32),
            pltpu.SemaphoreType.DMA,
        ],
    )


BN = 1000
GRID = N // BN


def _a_body(x_ref, w_ref, d0_ref, d1_ref, o0_ref, o1_ref, dis_ref):
    deg = d0_ref[...] + d1_ref[...] + 1.0
    dis = lax.rsqrt(deg)
    dis_ref[...] = dis
    z = jnp.dot(x_ref[...], w_ref[...], preferred_element_type=jnp.float32)
    z = z * dis
    h = z.shape[1] // 2
    o0_ref[...] = z[:, :h]
    o1_ref[...] = z[:, h:]


def _stage_a(x, w, d0, d1):
    return pl.pallas_call(
        _a_body,
        grid=(GRID,),
        in_specs=[
            pl.BlockSpec((BN, D_IN), lambda i: (i, 0)),
            pl.BlockSpec((D_IN, D_H), lambda i: (0, 0)),
            pl.BlockSpec((BN, 1), lambda i: (i, 0)),
            pl.BlockSpec((BN, 1), lambda i: (i, 0)),
        ],
        out_specs=[
            pl.BlockSpec((BN, D_H // 2), lambda i: (i, 0)),
            pl.BlockSpec((BN, D_H // 2), lambda i: (i, 0)),
            pl.BlockSpec((BN, 1), lambda i: (i, 0)),
        ],
        out_shape=[
            jax.ShapeDtypeStruct((N, D_H // 2), jnp.float32),
            jax.ShapeDtypeStruct((N, D_H // 2), jnp.float32),
            jax.ShapeDtypeStruct((N, 1), jnp.float32),
        ],
    )(x, w, d0, d1)


def _b_body(a0_ref, a1_ref, dis_ref, b_ref, w_ref, o0_ref, o1_ref):
    dis = dis_ref[...]
    u = jnp.concatenate([a0_ref[...] * dis, a1_ref[...] * dis], axis=1)
    u = jnp.maximum(u + b_ref[...], 0.0)
    z = jnp.dot(u, w_ref[...], preferred_element_type=jnp.float32) * dis
    h = z.shape[1] // 2
    o0_ref[...] = z[:, :h]
    o1_ref[...] = z[:, h:]


def _stage_b(a0, a1, dis2, b, w):
    dout = w.shape[1]
    return pl.pallas_call(
        _b_body,
        grid=(GRID,),
        in_specs=[
            pl.BlockSpec((BN, D_H // 2), lambda i: (i, 0)),
            pl.BlockSpec((BN, D_H // 2), lambda i: (i, 0)),
            pl.BlockSpec((BN, 1), lambda i: (i, 0)),
            pl.BlockSpec((1, D_H), lambda i: (0, 0)),
            pl.BlockSpec((D_H, dout), lambda i: (0, 0)),
        ],
        out_specs=[
            pl.BlockSpec((BN, dout // 2), lambda i: (i, 0)),
            pl.BlockSpec((BN, dout // 2), lambda i: (i, 0)),
        ],
        out_shape=[
            jax.ShapeDtypeStruct((N, dout // 2), jnp.float32),
            jax.ShapeDtypeStruct((N, dout // 2), jnp.float32),
        ],
    )(a0, a1, dis2, b, w)


def _c_body(a0_ref, a1_ref, dis_ref, b_ref, w_ref, o_ref):
    dis = dis_ref[...]
    u = jnp.concatenate([a0_ref[...] * dis, a1_ref[...] * dis], axis=1)
    u = jnp.maximum(u + b_ref[...], 0.0)
    o_ref[...] = jnp.dot(u, w_ref[...], preferred_element_type=jnp.float32) * dis


def _stage_c(a0, a1, dis2, b, w):
    dout = w.shape[1]
    return pl.pallas_call(
        _c_body,
        grid=(GRID,),
        in_specs=[
            pl.BlockSpec((BN, D_H // 2), lambda i: (i, 0)),
            pl.BlockSpec((BN, D_H // 2), lambda i: (i, 0)),
            pl.BlockSpec((BN, 1), lambda i: (i, 0)),
            pl.BlockSpec((1, D_H), lambda i: (0, 0)),
            pl.BlockSpec((D_H, dout), lambda i: (0, 0)),
        ],
        out_specs=pl.BlockSpec((BN, dout), lambda i: (i, 0)),
        out_shape=jax.ShapeDtypeStruct((N, dout), jnp.float32),
    )(a0, a1, dis2, b, w)


def _d_body(s0_ref, s1_ref, v_ref, dis_ref, bmu_ref, blv_ref, mu_ref, lv_ref):
    dis = dis_ref[...]
    m = (s0_ref[...] + s1_ref[...] - v_ref[...]) * dis
    mu_ref[...] = m[:, :D_Z] + bmu_ref[...]
    lv_ref[...] = m[:, D_Z:] + blv_ref[...]


def _stage_d(s0, s1, v, dis2, bmu, blv):
    return pl.pallas_call(
        _d_body,
        grid=(GRID,),
        in_specs=[
            pl.BlockSpec((BN, 2 * D_Z), lambda i: (i, 0)),
            pl.BlockSpec((BN, 2 * D_Z), lambda i: (i, 0)),
            pl.BlockSpec((BN, 2 * D_Z), lambda i: (i, 0)),
            pl.BlockSpec((BN, 1), lambda i: (i, 0)),
            pl.BlockSpec((1, D_Z), lambda i: (0, 0)),
            pl.BlockSpec((1, D_Z), lambda i: (0, 0)),
        ],
        out_specs=[
            pl.BlockSpec((BN, D_Z), lambda i: (i, 0)),
            pl.BlockSpec((BN, D_Z), lambda i: (i, 0)),
        ],
        out_shape=[
            jax.ShapeDtypeStruct((N, D_Z), jnp.float32),
            jax.ShapeDtypeStruct((N, D_Z), jnp.float32),
        ],
    )(s0, s1, v, dis2, bmu, blv)


def kernel(x, edge_index, W1, b1, W2, b2, Wmu, bmu, Wlv, blv):
    ei = edge_index.astype(jnp.int32).reshape(-1)
    degp = _deg_call()(ei)
    d0 = degp[:N].reshape(N, 1)
    d1 = degp[DEG_PAD:DEG_PAD + N].reshape(N, 1)

    h0, h1, dis2 = _stage_a(x, W1, d0, d1)
    a0, a1 = _prop_call(D_H // 2)(h0, h1, ei)
    g0, g1 = _stage_b(a0, a1, dis2, b1.reshape(1, -1), W2)
    p0, p1 = _prop_call(D_H // 2)(g0, g1, ei)
    wc = jnp.concatenate([Wmu, Wlv], axis=1)
    v = _stage_c(p0, p1, dis2, b2.reshape(1, -1), wc)
    s0, s1 = _prop_half_call()(v, ei)
    mu, lv = _stage_d(s0, s1, v, dis2,
                      bmu.reshape(1, -1), blv.reshape(1, -1))
    return (mu, lv)


# speedup vs baseline: 21.2667x; 1.9141x over previous
import functools

import jax
import jax.numpy as jnp
from jax import lax
from jax.experimental import pallas as pl
from jax.experimental.pallas import tpu as pltpu
from jax.experimental.pallas import tpu_sc as plsc

N = 10000
E = 160000
D_IN = 256
D_H = 256
D_Z = 64

NC = 2
NS = 16
CHUNK = 128
RCHUNKS = E // CHUNK
PCHUNKS = 1280
NKBUF = 40
SPLIT0 = NS * NKBUF
ROWS_PER_TILE = 624
TAIL_ROWS = N - NS * ROWS_PER_TILE
DEG_PAD = NS * 640

@functools.cache
def _mesh():
    return plsc.VectorSubcoreMesh(
        core_axis_name="c", subcore_axis_name="s",
        num_cores=NC, num_subcores=NS)


def _fill(ref, n, value, dtype):
    v = jnp.full((16,), value, dtype=dtype)
    for i in range(n // 16):
        ref[pl.ds(i * 16, 16)] = v


def _core_span(c, t):
    start = c * SPLIT0 + t * NKBUF
    end = jnp.where(c == 0, SPLIT0, RCHUNKS)
    n_k = jnp.minimum(NKBUF, end - start)
    return start, n_k


def _deg_body(dst_ref, deg_out, idx2, ones_v, zeros_v, deg_sh, ssem):
    c = lax.axis_index("c")
    t = lax.axis_index("s")
    _fill(ones_v, CHUNK, 1.0, jnp.float32)
    _fill(zeros_v, 640, 0.0, jnp.float32)
    pltpu.sync_copy(zeros_v, deg_sh.at[pl.ds(t * 640, 640)])

    start, n_k = _core_span(c, t)
    pltpu.sync_copy(dst_ref.at[pl.ds(start, NKBUF)], idx2)
    plsc.subcore_barrier()

    def body(k, carry):
        pltpu.async_copy(ones_v, deg_sh.at[idx2.at[k]], ssem, add=True)
        return carry

    lax.fori_loop(0, n_k, body, 0)

    def drain(k, carry):
        pltpu.make_async_copy(ones_v, deg_sh.at[idx2.at[k]], ssem).wait()
        return carry

    lax.fori_loop(0, n_k, drain, 0)
    plsc.subcore_barrier()
    pltpu.sync_copy(deg_sh.at[pl.ds(t * 640, 640)],
                    deg_out.at[pl.ds(c * DEG_PAD + t * 640, 640)])


@functools.cache
def _deg_call():
    return pl.kernel(
        _deg_body,
        out_type=jax.ShapeDtypeStruct((NC * DEG_PAD,), jnp.float32),
        mesh=_mesh(),
        scratch_types=[
            pltpu.VMEM((NKBUF, CHUNK), jnp.int32),
            pltpu.VMEM((CHUNK,), jnp.float32),
            pltpu.VMEM((640,), jnp.float32),
            pltpu.VMEM_SHARED((DEG_PAD,), jnp.float32),
            pltpu.SemaphoreType.DMA,
        ],
    )


def _copy_span(src, dst, t):
    r0 = t * ROWS_PER_TILE
    pltpu.sync_copy(src.at[pl.ds(r0, ROWS_PER_TILE)],
                    dst.at[pl.ds(r0, ROWS_PER_TILE)])

    @pl.when(t == NS - 1)
    def _():
        pltpu.sync_copy(src.at[pl.ds(NS * ROWS_PER_TILE, TAIL_ROWS)],
                        dst.at[pl.ds(NS * ROWS_PER_TILE, TAIL_ROWS)])


def _edge_sub(h, acc, src_ref, dst_ref, idx_s2, idx_d2, rows2, gsem,
              start, n_k):
    pltpu.sync_copy(src_ref.at[pl.ds(start, NKBUF)], idx_s2)
    pltpu.sync_copy(dst_ref.at[pl.ds(start, NKBUF)], idx_d2)

    def gather(k, b):
        return pltpu.async_copy(h.at[idx_s2.at[k]], rows2.at[b], gsem)

    def gwait(b):
        pltpu.make_async_copy(h.at[idx_s2.at[0]], rows2.at[b], gsem).wait()

    def scat(k, b):
        pltpu.sync_copy(rows2.at[b], acc.at[idx_d2.at[k]], add=True)

    @pl.when(n_k > 0)
    def _():
        gather(0, 0)

        def pair(p, carry):
            k0 = 2 * p

            @pl.when(k0 + 1 < n_k)
            def _():
                gather(k0 + 1, 1)

            gwait(0)
            scat(k0, 0)

            @pl.when(k0 + 2 < n_k)
            def _():
                gather(k0 + 2, 0)

            @pl.when(k0 + 1 < n_k)
            def _():
                gwait(1)
                scat(k0 + 1, 1)

            return carry

        lax.fori_loop(0, (n_k + 1) // 2, pair, 0)


def _prop_core(h, o, src_ref, dst_ref, idx_s2, idx_d2, rows2, acc, gsem, t):
    _copy_span(h, acc, t)
    plsc.subcore_barrier()
    for s in range(2):
        start = (2 * t + s) * NKBUF
        n_k = jnp.minimum(NKBUF, RCHUNKS - start)
        _edge_sub(h, acc, src_ref, dst_ref, idx_s2, idx_d2, rows2, gsem,
                  start, n_k)
    plsc.subcore_barrier()
    _copy_span(acc, o, t)


def _prop_body(h0, h1, src_ref, dst_ref, o0, o1,
               idx_s2, idx_d2, rows2, acc, gsem):
    c = lax.axis_index("c")
    t = lax.axis_index("s")

    @pl.when(c == 0)
    def _():
        _prop_core(h0, o0, src_ref, dst_ref,
                   idx_s2, idx_d2, rows2, acc, gsem, t)

    @pl.when(c == 1)
    def _():
        _prop_core(h1, o1, src_ref, dst_ref,
                   idx_s2, idx_d2, rows2, acc, gsem, t)


def _prop_half_body(v, src_ref, dst_ref, o0, o1,
                    idx_s2, idx_d2, rows2, acc, gsem):
    c = lax.axis_index("c")
    t = lax.axis_index("s")
    _copy_span(v, acc, t)
    plsc.subcore_barrier()
    start, n_k = _core_span(c, t)
    _edge_sub(v, acc, src_ref, dst_ref, idx_s2, idx_d2, rows2, gsem,
              start, n_k)
    plsc.subcore_barrier()

    @pl.when(c == 0)
    def _():
        _copy_span(acc, o0, t)

    @pl.when(c == 1)
    def _():
        _copy_span(acc, o1, t)


@functools.cache
def _prop_half_call():
    return pl.kernel(
        _prop_half_body,
        out_type=(jax.ShapeDtypeStruct((N, 2 * D_Z), jnp.float32),
                  jax.ShapeDtypeStruct((N, 2 * D_Z), jnp.float32)),
        mesh=_mesh(),
        scratch_types=[
            pltpu.VMEM((NKBUF, CHUNK), jnp.int32),
            pltpu.VMEM((NKBUF, CHUNK), jnp.int32),
            pltpu.VMEM((2, CHUNK, 2 * D_Z), jnp.float32),
            pltpu.VMEM_SHARED((N, 2 * D_Z), jnp.float32),
            pltpu.SemaphoreType.DMA,
        ],
    )


@functools.cache
def _prop_call(w):
    return pl.kernel(
        _prop_body,
        out_type=(jax.ShapeDtypeStruct((N, w), jnp.float32),
                  jax.ShapeDtypeStruct((N, w), jnp.float32)),
        mesh=_mesh(),
        scratch_types=[
            pltpu.VMEM((NKBUF, CHUNK), jnp.int32),
            pltpu.VMEM((NKBUF, CHUNK), jnp.int32),
            pltpu.VMEM((2, CHUNK, w), jnp.float32),
            pltpu.VMEM_SHARED((N, w), jnp.float32),
            pltpu.SemaphoreType.DMA,
        ],
    )


BN = 1000
GRID = N // BN


def _a_body(x_ref, w_ref, d0_ref, d1_ref, o0_ref, o1_ref, dis_ref):
    deg = d0_ref[...] + d1_ref[...] + 1.0
    dis = lax.rsqrt(deg)
    dis_ref[...] = dis
    z = jnp.dot(x_ref[...], w_ref[...], preferred_element_type=jnp.float32)
    z = z * dis
    h = z.shape[1] // 2
    o0_ref[...] = z[:, :h]
    o1_ref[...] = z[:, h:]


def _stage_a(x, w, d0, d1):
    return pl.pallas_call(
        _a_body,
        grid=(GRID,),
        in_specs=[
            pl.BlockSpec((BN, D_IN), lambda i: (i, 0)),
            pl.BlockSpec((D_IN, D_H), lambda i: (0, 0)),
            pl.BlockSpec((BN, 1), lambda i: (i, 0)),
            pl.BlockSpec((BN, 1), lambda i: (i, 0)),
        ],
        out_specs=[
            pl.BlockSpec((BN, D_H // 2), lambda i: (i, 0)),
            pl.BlockSpec((BN, D_H // 2), lambda i: (i, 0)),
            pl.BlockSpec((BN, 1), lambda i: (i, 0)),
        ],
        out_shape=[
            jax.ShapeDtypeStruct((N, D_H // 2), jnp.float32),
            jax.ShapeDtypeStruct((N, D_H // 2), jnp.float32),
            jax.ShapeDtypeStruct((N, 1), jnp.float32),
        ],
    )(x, w, d0, d1)


def _b_body(a0_ref, a1_ref, dis_ref, b_ref, w_ref, o0_ref, o1_ref):
    dis = dis_ref[...]
    u = jnp.concatenate([a0_ref[...] * dis, a1_ref[...] * dis], axis=1)
    u = jnp.maximum(u + b_ref[...], 0.0)
    z = jnp.dot(u, w_ref[...], preferred_element_type=jnp.float32) * dis
    h = z.shape[1] // 2
    o0_ref[...] = z[:, :h]
    o1_ref[...] = z[:, h:]


def _stage_b(a0, a1, dis2, b, w):
    dout = w.shape[1]
    return pl.pallas_call(
        _b_body,
        grid=(GRID,),
        in_specs=[
            pl.BlockSpec((BN, D_H // 2), lambda i: (i, 0)),
            pl.BlockSpec((BN, D_H // 2), lambda i: (i, 0)),
            pl.BlockSpec((BN, 1), lambda i: (i, 0)),
            pl.BlockSpec((1, D_H), lambda i: (0, 0)),
            pl.BlockSpec((D_H, dout), lambda i: (0, 0)),
        ],
        out_specs=[
            pl.BlockSpec((BN, dout // 2), lambda i: (i, 0)),
            pl.BlockSpec((BN, dout // 2), lambda i: (i, 0)),
        ],
        out_shape=[
            jax.ShapeDtypeStruct((N, dout // 2), jnp.float32),
            jax.ShapeDtypeStruct((N, dout // 2), jnp.float32),
        ],
    )(a0, a1, dis2, b, w)


def _c_body(a0_ref, a1_ref, dis_ref, b_ref, w_ref, o_ref):
    dis = dis_ref[...]
    u = jnp.concatenate([a0_ref[...] * dis, a1_ref[...] * dis], axis=1)
    u = jnp.maximum(u + b_ref[...], 0.0)
    o_ref[...] = jnp.dot(u, w_ref[...], preferred_element_type=jnp.float32) * dis


def _stage_c(a0, a1, dis2, b, w):
    dout = w.shape[1]
    return pl.pallas_call(
        _c_body,
        grid=(GRID,),
        in_specs=[
            pl.BlockSpec((BN, D_H // 2), lambda i: (i, 0)),
            pl.BlockSpec((BN, D_H // 2), lambda i: (i, 0)),
            pl.BlockSpec((BN, 1), lambda i: (i, 0)),
            pl.BlockSpec((1, D_H), lambda i: (0, 0)),
            pl.BlockSpec((D_H, dout), lambda i: (0, 0)),
        ],
        out_specs=pl.BlockSpec((BN, dout), lambda i: (i, 0)),
        out_shape=jax.ShapeDtypeStruct((N, dout), jnp.float32),
    )(a0, a1, dis2, b, w)


def _d_body(s0_ref, s1_ref, v_ref, dis_ref, bmu_ref, blv_ref, mu_ref, lv_ref):
    dis = dis_ref[...]
    m = (s0_ref[...] + s1_ref[...] - v_ref[...]) * dis
    mu_ref[...] = m[:, :D_Z] + bmu_ref[...]
    lv_ref[...] = m[:, D_Z:] + blv_ref[...]


def _stage_d(s0, s1, v, dis2, bmu, blv):
    return pl.pallas_call(
        _d_body,
        grid=(GRID,),
        in_specs=[
            pl.BlockSpec((BN, 2 * D_Z), lambda i: (i, 0)),
            pl.BlockSpec((BN, 2 * D_Z), lambda i: (i, 0)),
            pl.BlockSpec((BN, 2 * D_Z), lambda i: (i, 0)),
            pl.BlockSpec((BN, 1), lambda i: (i, 0)),
            pl.BlockSpec((1, D_Z), lambda i: (0, 0)),
            pl.BlockSpec((1, D_Z), lambda i: (0, 0)),
        ],
        out_specs=[
            pl.BlockSpec((BN, D_Z), lambda i: (i, 0)),
            pl.BlockSpec((BN, D_Z), lambda i: (i, 0)),
        ],
        out_shape=[
            jax.ShapeDtypeStruct((N, D_Z), jnp.float32),
            jax.ShapeDtypeStruct((N, D_Z), jnp.float32),
        ],
    )(s0, s1, v, dis2, bmu, blv)


def kernel(x, edge_index, W1, b1, W2, b2, Wmu, bmu, Wlv, blv):
    eii = edge_index.astype(jnp.int32)
    pad = PCHUNKS * CHUNK - E
    zpad = jnp.zeros((pad,), jnp.int32)
    src2 = jnp.concatenate([eii[0], zpad]).reshape(PCHUNKS, CHUNK)
    dst2 = jnp.concatenate([eii[1], zpad]).reshape(PCHUNKS, CHUNK)
    degp = _deg_call()(dst2)
    d0 = degp[:N].reshape(N, 1)
    d1 = degp[DEG_PAD:DEG_PAD + N].reshape(N, 1)

    h0, h1, dis2 = _stage_a(x, W1, d0, d1)
    a0, a1 = _prop_call(D_H // 2)(h0, h1, src2, dst2)
    g0, g1 = _stage_b(a0, a1, dis2, b1.reshape(1, -1), W2)
    p0, p1 = _prop_call(D_H // 2)(g0, g1, src2, dst2)
    wc = jnp.concatenate([Wmu, Wlv], axis=1)
    v = _stage_c(p0, p1, dis2, b2.reshape(1, -1), wc)
    s0, s1 = _prop_half_call()(v, src2, dst2)
    mu, lv = _stage_d(s0, s1, v, dis2,
                      bmu.reshape(1, -1), blv.reshape(1, -1))
    return (mu, lv)


# speedup vs baseline: 22.1892x; 1.0434x over previous
import functools

import jax
import jax.numpy as jnp
from jax import lax
from jax.experimental import pallas as pl
from jax.experimental.pallas import tpu as pltpu
from jax.experimental.pallas import tpu_sc as plsc

N = 10000
E = 160000
D_IN = 256
D_H = 256
D_Z = 64

NC = 2
NS = 16
CHUNK = 128
RCHUNKS = E // CHUNK
PCHUNKS = 1280
NKBUF = 40
SPLIT0 = NS * NKBUF
ROWS_PER_TILE = 624
TAIL_ROWS = N - NS * ROWS_PER_TILE
DEG_PAD = NS * 640

@functools.cache
def _mesh():
    return plsc.VectorSubcoreMesh(
        core_axis_name="c", subcore_axis_name="s",
        num_cores=NC, num_subcores=NS)


def _fill(ref, n, value, dtype):
    v = jnp.full((16,), value, dtype=dtype)
    for i in range(n // 16):
        ref[pl.ds(i * 16, 16)] = v


def _core_span(c, t):
    start = c * SPLIT0 + t * NKBUF
    end = jnp.where(c == 0, SPLIT0, RCHUNKS)
    n_k = jnp.minimum(NKBUF, end - start)
    return start, n_k


def _deg_body(dst_ref, deg_out, idx2, ones_v, zeros_v, deg_sh, ssem):
    c = lax.axis_index("c")
    t = lax.axis_index("s")
    start, n_k = _core_span(c, t)
    ia = pltpu.async_copy(dst_ref.at[pl.ds(start, NKBUF)], idx2, ssem)
    _fill(ones_v, CHUNK, 1.0, jnp.float32)
    _fill(zeros_v, 640, 0.0, jnp.float32)
    pltpu.sync_copy(zeros_v, deg_sh.at[pl.ds(t * 640, 640)])
    ia.wait()
    plsc.subcore_barrier()

    def body(k, carry):
        pltpu.async_copy(ones_v, deg_sh.at[idx2.at[k]], ssem, add=True)
        return carry

    lax.fori_loop(0, n_k, body, 0)

    def drain(k, carry):
        pltpu.make_async_copy(ones_v, deg_sh.at[idx2.at[k]], ssem).wait()
        return carry

    lax.fori_loop(0, n_k, drain, 0)
    plsc.subcore_barrier()
    pltpu.sync_copy(deg_sh.at[pl.ds(t * 640, 640)],
                    deg_out.at[pl.ds(c * DEG_PAD + t * 640, 640)])


@functools.cache
def _deg_call():
    return pl.kernel(
        _deg_body,
        out_type=jax.ShapeDtypeStruct((NC * DEG_PAD,), jnp.float32),
        mesh=_mesh(),
        scratch_types=[
            pltpu.VMEM((NKBUF, CHUNK), jnp.int32),
            pltpu.VMEM((CHUNK,), jnp.float32),
            pltpu.VMEM((640,), jnp.float32),
            pltpu.VMEM_SHARED((DEG_PAD,), jnp.float32),
            pltpu.SemaphoreType.DMA,
        ],
    )


def _copy_span(src, dst, t):
    r0 = t * ROWS_PER_TILE
    pltpu.sync_copy(src.at[pl.ds(r0, ROWS_PER_TILE)],
                    dst.at[pl.ds(r0, ROWS_PER_TILE)])

    @pl.when(t == NS - 1)
    def _():
        pltpu.sync_copy(src.at[pl.ds(NS * ROWS_PER_TILE, TAIL_ROWS)],
                        dst.at[pl.ds(NS * ROWS_PER_TILE, TAIL_ROWS)])


def _load_idx(src_ref, dst_ref, idx_s2, idx_d2, gsem, start):
    a = pltpu.async_copy(src_ref.at[pl.ds(start, NKBUF)], idx_s2, gsem)
    b = pltpu.async_copy(dst_ref.at[pl.ds(start, NKBUF)], idx_d2, gsem)
    return a, b


def _edge_sub(h, acc, idx_s2, idx_d2, rows2, gsem, n_k, skip_prolog=False):
    def gather(k, b):
        return pltpu.async_copy(h.at[idx_s2.at[k]], rows2.at[b], gsem)

    def gwait(b):
        pltpu.make_async_copy(h.at[idx_s2.at[0]], rows2.at[b], gsem).wait()

    def scat(k, b):
        pltpu.sync_copy(rows2.at[b], acc.at[idx_d2.at[k]], add=True)

    @pl.when(n_k > 0)
    def _():
        if not skip_prolog:
            gather(0, 0)

        def pair(p, carry):
            k0 = 2 * p

            @pl.when(k0 + 1 < n_k)
            def _():
                gather(k0 + 1, 1)

            gwait(0)
            scat(k0, 0)

            @pl.when(k0 + 2 < n_k)
            def _():
                gather(k0 + 2, 0)

            @pl.when(k0 + 1 < n_k)
            def _():
                gwait(1)
                scat(k0 + 1, 1)

            return carry

        lax.fori_loop(0, (n_k + 1) // 2, pair, 0)

    return gather


def _prop_core(h, o, src_ref, dst_ref, idx_s2, idx_d2, rows2, acc, gsem, t):
    ia, ib = _load_idx(src_ref, dst_ref, idx_s2, idx_d2, gsem, 2 * t * NKBUF)
    _copy_span(h, acc, t)
    ia.wait()
    ib.wait()
    pltpu.async_copy(h.at[idx_s2.at[0]], rows2.at[0], gsem)
    plsc.subcore_barrier()
    _edge_sub(h, acc, idx_s2, idx_d2, rows2, gsem,
              jnp.minimum(NKBUF, RCHUNKS - 2 * t * NKBUF), skip_prolog=True)
    start2 = (2 * t + 1) * NKBUF
    ia, ib = _load_idx(src_ref, dst_ref, idx_s2, idx_d2, gsem, start2)
    ia.wait()
    ib.wait()
    _edge_sub(h, acc, idx_s2, idx_d2, rows2, gsem,
              jnp.minimum(NKBUF, RCHUNKS - start2))
    plsc.subcore_barrier()
    _copy_span(acc, o, t)


def _prop_body(h0, h1, src_ref, dst_ref, o0, o1,
               idx_s2, idx_d2, rows2, acc, gsem):
    c = lax.axis_index("c")
    t = lax.axis_index("s")

    @pl.when(c == 0)
    def _():
        _prop_core(h0, o0, src_ref, dst_ref,
                   idx_s2, idx_d2, rows2, acc, gsem, t)

    @pl.when(c == 1)
    def _():
        _prop_core(h1, o1, src_ref, dst_ref,
                   idx_s2, idx_d2, rows2, acc, gsem, t)


def _prop_half_body(v, src_ref, dst_ref, o0, o1,
                    idx_s2, idx_d2, rows2, acc, gsem):
    c = lax.axis_index("c")
    t = lax.axis_index("s")
    start, n_k = _core_span(c, t)
    ia, ib = _load_idx(src_ref, dst_ref, idx_s2, idx_d2, gsem, start)
    _copy_span(v, acc, t)
    ia.wait()
    ib.wait()
    pltpu.async_copy(v.at[idx_s2.at[0]], rows2.at[0], gsem)
    plsc.subcore_barrier()
    _edge_sub(v, acc, idx_s2, idx_d2, rows2, gsem, n_k, skip_prolog=True)
    plsc.subcore_barrier()

    @pl.when(c == 0)
    def _():
        _copy_span(acc, o0, t)

    @pl.when(c == 1)
    def _():
        _copy_span(acc, o1, t)


@functools.cache
def _prop_half_call():
    return pl.kernel(
        _prop_half_body,
        out_type=(jax.ShapeDtypeStruct((N, 2 * D_Z), jnp.float32),
                  jax.ShapeDtypeStruct((N, 2 * D_Z), jnp.float32)),
        mesh=_mesh(),
        scratch_types=[
            pltpu.VMEM((NKBUF, CHUNK), jnp.int32),
            pltpu.VMEM((NKBUF, CHUNK), jnp.int32),
            pltpu.VMEM((2, CHUNK, 2 * D_Z), jnp.float32),
            pltpu.VMEM_SHARED((N, 2 * D_Z), jnp.float32),
            pltpu.SemaphoreType.DMA,
        ],
    )


@functools.cache
def _prop_call(w):
    return pl.kernel(
        _prop_body,
        out_type=(jax.ShapeDtypeStruct((N, w), jnp.float32),
                  jax.ShapeDtypeStruct((N, w), jnp.float32)),
        mesh=_mesh(),
        scratch_types=[
            pltpu.VMEM((NKBUF, CHUNK), jnp.int32),
            pltpu.VMEM((NKBUF, CHUNK), jnp.int32),
            pltpu.VMEM((2, CHUNK, w), jnp.float32),
            pltpu.VMEM_SHARED((N, w), jnp.float32),
            pltpu.SemaphoreType.DMA,
        ],
    )


BN = 2000
GRID = N // BN


def _a_body(x_ref, w_ref, d0_ref, d1_ref, o0_ref, o1_ref, dis_ref):
    deg = d0_ref[...] + d1_ref[...] + 1.0
    dis = lax.rsqrt(deg)
    dis_ref[...] = dis
    z = jnp.dot(x_ref[...], w_ref[...], preferred_element_type=jnp.float32)
    z = z * dis
    h = z.shape[1] // 2
    o0_ref[...] = z[:, :h]
    o1_ref[...] = z[:, h:]


def _stage_a(x, w, d0, d1):
    return pl.pallas_call(
        _a_body,
        grid=(GRID,),
        in_specs=[
            pl.BlockSpec((BN, D_IN), lambda i: (i, 0)),
            pl.BlockSpec((D_IN, D_H), lambda i: (0, 0)),
            pl.BlockSpec((BN, 1), lambda i: (i, 0)),
            pl.BlockSpec((BN, 1), lambda i: (i, 0)),
        ],
        out_specs=[
            pl.BlockSpec((BN, D_H // 2), lambda i: (i, 0)),
            pl.BlockSpec((BN, D_H // 2), lambda i: (i, 0)),
            pl.BlockSpec((BN, 1), lambda i: (i, 0)),
        ],
        out_shape=[
            jax.ShapeDtypeStruct((N, D_H // 2), jnp.float32),
            jax.ShapeDtypeStruct((N, D_H // 2), jnp.float32),
            jax.ShapeDtypeStruct((N, 1), jnp.float32),
        ],
    )(x, w, d0, d1)


def _b_body(a0_ref, a1_ref, dis_ref, b_ref, w_ref, o0_ref, o1_ref):
    dis = dis_ref[...]
    u = jnp.concatenate([a0_ref[...] * dis, a1_ref[...] * dis], axis=1)
    u = jnp.maximum(u + b_ref[...], 0.0)
    z = jnp.dot(u, w_ref[...], preferred_element_type=jnp.float32) * dis
    h = z.shape[1] // 2
    o0_ref[...] = z[:, :h]
    o1_ref[...] = z[:, h:]


def _stage_b(a0, a1, dis2, b, w):
    dout = w.shape[1]
    return pl.pallas_call(
        _b_body,
        grid=(GRID,),
        in_specs=[
            pl.BlockSpec((BN, D_H // 2), lambda i: (i, 0)),
            pl.BlockSpec((BN, D_H // 2), lambda i: (i, 0)),
            pl.BlockSpec((BN, 1), lambda i: (i, 0)),
            pl.BlockSpec((1, D_H), lambda i: (0, 0)),
            pl.BlockSpec((D_H, dout), lambda i: (0, 0)),
        ],
        out_specs=[
            pl.BlockSpec((BN, dout // 2), lambda i: (i, 0)),
            pl.BlockSpec((BN, dout // 2), lambda i: (i, 0)),
        ],
        out_shape=[
            jax.ShapeDtypeStruct((N, dout // 2), jnp.float32),
            jax.ShapeDtypeStruct((N, dout // 2), jnp.float32),
        ],
    )(a0, a1, dis2, b, w)


def _c_body(a0_ref, a1_ref, dis_ref, b_ref, w_ref, o_ref):
    dis = dis_ref[...]
    u = jnp.concatenate([a0_ref[...] * dis, a1_ref[...] * dis], axis=1)
    u = jnp.maximum(u + b_ref[...], 0.0)
    o_ref[...] = jnp.dot(u, w_ref[...], preferred_element_type=jnp.float32) * dis


def _stage_c(a0, a1, dis2, b, w):
    dout = w.shape[1]
    return pl.pallas_call(
        _c_body,
        grid=(GRID,),
        in_specs=[
            pl.BlockSpec((BN, D_H // 2), lambda i: (i, 0)),
            pl.BlockSpec((BN, D_H // 2), lambda i: (i, 0)),
            pl.BlockSpec((BN, 1), lambda i: (i, 0)),
            pl.BlockSpec((1, D_H), lambda i: (0, 0)),
            pl.BlockSpec((D_H, dout), lambda i: (0, 0)),
        ],
        out_specs=pl.BlockSpec((BN, dout), lambda i: (i, 0)),
        out_shape=jax.ShapeDtypeStruct((N, dout), jnp.float32),
    )(a0, a1, dis2, b, w)


def _d_body(s0_ref, s1_ref, v_ref, dis_ref, bmu_ref, blv_ref, mu_ref, lv_ref):
    dis = dis_ref[...]
    m = (s0_ref[...] + s1_ref[...] - v_ref[...]) * dis
    mu_ref[...] = m[:, :D_Z] + bmu_ref[...]
    lv_ref[...] = m[:, D_Z:] + blv_ref[...]


def _stage_d(s0, s1, v, dis2, bmu, blv):
    return pl.pallas_call(
        _d_body,
        grid=(GRID,),
        in_specs=[
            pl.BlockSpec((BN, 2 * D_Z), lambda i: (i, 0)),
            pl.BlockSpec((BN, 2 * D_Z), lambda i: (i, 0)),
            pl.BlockSpec((BN, 2 * D_Z), lambda i: (i, 0)),
            pl.BlockSpec((BN, 1), lambda i: (i, 0)),
            pl.BlockSpec((1, D_Z), lambda i: (0, 0)),
            pl.BlockSpec((1, D_Z), lambda i: (0, 0)),
        ],
        out_specs=[
            pl.BlockSpec((BN, D_Z), lambda i: (i, 0)),
            pl.BlockSpec((BN, D_Z), lambda i: (i, 0)),
        ],
        out_shape=[
            jax.ShapeDtypeStruct((N, D_Z), jnp.float32),
            jax.ShapeDtypeStruct((N, D_Z), jnp.float32),
        ],
    )(s0, s1, v, dis2, bmu, blv)


def kernel(x, edge_index, W1, b1, W2, b2, Wmu, bmu, Wlv, blv):
    eii = edge_index.astype(jnp.int32)
    pad = PCHUNKS * CHUNK - E
    zpad = jnp.zeros((pad,), jnp.int32)
    src2 = jnp.concatenate([eii[0], zpad]).reshape(PCHUNKS, CHUNK)
    dst2 = jnp.concatenate([eii[1], zpad]).reshape(PCHUNKS, CHUNK)
    degp = _deg_call()(dst2)
    d0 = degp[:N].reshape(N, 1)
    d1 = degp[DEG_PAD:DEG_PAD + N].reshape(N, 1)

    h0, h1, dis2 = _stage_a(x, W1, d0, d1)
    a0, a1 = _prop_call(D_H // 2)(h0, h1, src2, dst2)
    g0, g1 = _stage_b(a0, a1, dis2, b1.reshape(1, -1), W2)
    p0, p1 = _prop_call(D_H // 2)(g0, g1, src2, dst2)
    wc = jnp.concatenate([Wmu, Wlv], axis=1)
    v = _stage_c(p0, p1, dis2, b2.reshape(1, -1), wc)
    s0, s1 = _prop_half_call()(v, src2, dst2)
    mu, lv = _stage_d(s0, s1, v, dis2,
                      bmu.reshape(1, -1), blv.reshape(1, -1))
    return (mu, lv)
